# Initial kernel scaffold; baseline (speedup 1.0000x reference)
import jax
import jax.numpy as jnp
from jax.experimental import pallas as pl


def kernel(former_proto_list, former_proto_label, tf_feat_list, tf_label_list):
    raise NotImplementedError("write your pallas kernel here")


# speedup vs baseline: 4.2002x; 4.2002x over previous
import functools

import jax
import jax.numpy as jnp
from jax import lax
from jax.experimental import pallas as pl
from jax.experimental.pallas import tpu as pltpu
from jax.experimental.pallas import tpu_sc as plsc

NC = 2
NS = 16
NW = NC * NS
CHUNK = 80


def _sc_partial_sums(feats, labels1d, zeros_acc, zeros_cnt, ones_cnt, c):
    n, d = feats.shape
    nblk = n // CHUNK
    blk_per_tile = nblk // NW
    mesh = plsc.VectorSubcoreMesh(
        core_axis_name="c", subcore_axis_name="s", num_cores=NC, num_subcores=NS)

    @functools.partial(
        pl.kernel,
        out_type=[
            jax.ShapeDtypeStruct((NC, NS * c, d), jnp.float32),
            jax.ShapeDtypeStruct((NC, NS * c, 128), jnp.float32),
        ],
        mesh=mesh,
        scratch_types=[
            pltpu.VMEM((CHUNK,), jnp.int32),
            pltpu.VMEM((CHUNK, d), jnp.float32),
            pltpu.VMEM((CHUNK, 128), jnp.float32),
            pltpu.VMEM_SHARED((NS * c, d), jnp.float32),
            pltpu.VMEM_SHARED((NS * c, 128), jnp.float32),
        ],
    )
    def run(feats_hbm, labels_hbm, zacc_hbm, zcnt_hbm, ones_hbm,
            out_sums, out_cnts, idx_v, buf, ones_v, shared_acc, shared_cnt):
        cid = lax.axis_index("c")
        sid = lax.axis_index("s")
        wid = cid * NS + sid
        @pl.when(sid == 0)
        def _():
            pltpu.sync_copy(zacc_hbm, shared_acc)
            pltpu.sync_copy(zcnt_hbm, shared_cnt)

        pltpu.sync_copy(ones_hbm, ones_v)
        plsc.subcore_barrier()

        def body(j, carry):
            blk = wid * blk_per_tile + j
            pltpu.sync_copy(labels_hbm.at[pl.ds(blk * CHUNK, CHUNK)], idx_v)
            pltpu.sync_copy(feats_hbm.at[pl.ds(blk * CHUNK, CHUNK)], buf)
            pltpu.sync_copy(buf, shared_acc.at[idx_v], add=True)
            pltpu.sync_copy(ones_v, shared_cnt.at[idx_v], add=True)
            return carry

        lax.fori_loop(0, blk_per_tile, body, 0)
        plsc.subcore_barrier()

        @pl.when(sid == 0)
        def _():
            pltpu.sync_copy(shared_acc, out_sums.at[cid])
            pltpu.sync_copy(shared_cnt, out_cnts.at[cid])

    return run(feats, labels1d, zeros_acc, zeros_cnt, ones_cnt)


def _combine_kernel(psums_ref, pcnts_ref, proto_ref, plabel_ref, out_ref):
    sums = jnp.sum(psums_ref[...], axis=0)
    cnts = jnp.sum(pcnts_ref[...], axis=0)[:, :1]
    c = sums.shape[0]
    d = sums.shape[1]
    labels = plabel_ref[...]
    onehot = (labels[:, None] ==
              lax.broadcasted_iota(jnp.int32, (c, c), 1)).astype(jnp.float32)
    sums_sel = jnp.dot(onehot, sums, preferred_element_type=jnp.float32)
    cnts_sel = jnp.dot(onehot, cnts, preferred_element_type=jnp.float32)
    safe = jnp.maximum(cnts_sel, 1.0)
    proto_cur = sums_sel / safe
    present = (cnts_sel > 0).astype(jnp.float32)
    sq = (proto_ref[...] - proto_cur) ** 2
    per_class = jnp.sum(sq, axis=1, keepdims=True) * present
    denom = jnp.maximum(jnp.sum(present) * d, 1.0)
    out_ref[0, 0] = jnp.sum(per_class) / denom


def kernel(former_proto_list, former_proto_label, tf_feat_list, tf_label_list):
    n, d = tf_feat_list.shape
    c = former_proto_list.shape[0]
    nblk = n // CHUNK
    blk_per_tile = nblk // NW
    rows_per_tile = n // NW
    sid = (jnp.arange(n, dtype=jnp.int32) // rows_per_tile) % NS
    labels1d = tf_label_list.astype(jnp.int32) + sid * c
    zeros_acc = jnp.zeros((NS * c, d), jnp.float32)
    zeros_cnt = jnp.zeros((NS * c, 128), jnp.float32)
    ones_cnt = jnp.ones((CHUNK, 128), jnp.float32)
    psums, pcnts = _sc_partial_sums(
        tf_feat_list, labels1d, zeros_acc, zeros_cnt, ones_cnt, c)
    psums = psums.reshape(NW, c, d)
    pcnts = pcnts.reshape(NW, c, 128)
    out = pl.pallas_call(
        _combine_kernel,
        out_shape=jax.ShapeDtypeStruct((1, 1), jnp.float32),
        out_specs=pl.BlockSpec(memory_space=pltpu.SMEM),
    )(psums, pcnts, former_proto_list, former_proto_label.astype(jnp.int32))
    return out[0, 0]


# speedup vs baseline: 7.7050x; 1.8344x over previous
import functools

import jax
import jax.numpy as jnp
from jax import lax
from jax.experimental import pallas as pl
from jax.experimental.pallas import tpu as pltpu
from jax.experimental.pallas import tpu_sc as plsc

NC = 2
NS = 16
NW = NC * NS
CHUNK = 80


def _sc_partial_sums(feats, labels1d, zeros_acc, zeros_cnt, ones_cnt, c):
    n, d = feats.shape
    nblk = n // CHUNK
    blk_per_tile = nblk // NW
    mesh = plsc.VectorSubcoreMesh(
        core_axis_name="c", subcore_axis_name="s", num_cores=NC, num_subcores=NS)

    @functools.partial(
        pl.kernel,
        out_type=[
            jax.ShapeDtypeStruct((NC, NS * c, d), jnp.float32),
            jax.ShapeDtypeStruct((NC, NS * c, 128), jnp.float32),
        ],
        mesh=mesh,
        scratch_types=[
            pltpu.VMEM((2, CHUNK), jnp.int32),
            pltpu.VMEM((2, CHUNK, d), jnp.float32),
            pltpu.VMEM((CHUNK, 128), jnp.float32),
            pltpu.VMEM_SHARED((NS * c, d), jnp.float32),
            pltpu.VMEM_SHARED((NS * c, 128), jnp.float32),
            pltpu.SemaphoreType.DMA((2,)),
            pltpu.SemaphoreType.DMA((2,)),
        ],
    )
    def run(feats_hbm, labels_hbm, zacc_hbm, zcnt_hbm, ones_hbm,
            out_sums, out_cnts, idx2, buf2, ones_v, shared_acc, shared_cnt,
            lsem, fsem):
        cid = lax.axis_index("c")
        sid = lax.axis_index("s")
        wid = cid * NS + sid
        @pl.when(sid == 0)
        def _():
            pltpu.sync_copy(zacc_hbm, shared_acc)
            pltpu.sync_copy(zcnt_hbm, shared_cnt)

        pltpu.sync_copy(ones_hbm, ones_v)
        plsc.subcore_barrier()

        def start(j):
            p = lax.rem(j, 2)
            blk = wid * blk_per_tile + j
            pltpu.async_copy(
                labels_hbm.at[pl.ds(blk * CHUNK, CHUNK)], idx2.at[p],
                lsem.at[p])
            pltpu.async_copy(
                feats_hbm.at[pl.ds(blk * CHUNK, CHUNK)], buf2.at[p],
                fsem.at[p])

        start(0)

        def body(j, carry):
            p = lax.rem(j, 2)
            pltpu.make_async_copy(
                labels_hbm.at[pl.ds(0, CHUNK)], idx2.at[p], lsem.at[p]).wait()
            pltpu.make_async_copy(
                feats_hbm.at[pl.ds(0, CHUNK)], buf2.at[p], fsem.at[p]).wait()

            @pl.when(j < blk_per_tile - 1)
            def _():
                start(j + 1)

            pltpu.sync_copy(buf2.at[p], shared_acc.at[idx2.at[p]], add=True)
            pltpu.sync_copy(ones_v, shared_cnt.at[idx2.at[p]], add=True)
            return carry

        lax.fori_loop(0, blk_per_tile, body, 0)
        plsc.subcore_barrier()

        @pl.when(sid == 0)
        def _():
            pltpu.sync_copy(shared_acc, out_sums.at[cid])
            pltpu.sync_copy(shared_cnt, out_cnts.at[cid])

    return run(feats, labels1d, zeros_acc, zeros_cnt, ones_cnt)


def _combine_kernel(psums_ref, pcnts_ref, proto_ref, plabel_ref, out_ref):
    sums = jnp.sum(psums_ref[...], axis=0)
    cnts = jnp.sum(pcnts_ref[...], axis=0)[:, :1]
    c = sums.shape[0]
    d = sums.shape[1]
    labels = plabel_ref[...]
    onehot = (labels[:, None] ==
              lax.broadcasted_iota(jnp.int32, (c, c), 1)).astype(jnp.float32)
    sums_sel = jnp.dot(onehot, sums, preferred_element_type=jnp.float32)
    cnts_sel = jnp.dot(onehot, cnts, preferred_element_type=jnp.float32)
    safe = jnp.maximum(cnts_sel, 1.0)
    proto_cur = sums_sel / safe
    present = (cnts_sel > 0).astype(jnp.float32)
    sq = (proto_ref[...] - proto_cur) ** 2
    per_class = jnp.sum(sq, axis=1, keepdims=True) * present
    denom = jnp.maximum(jnp.sum(present) * d, 1.0)
    out_ref[0, 0] = jnp.sum(per_class) / denom


def kernel(former_proto_list, former_proto_label, tf_feat_list, tf_label_list):
    n, d = tf_feat_list.shape
    c = former_proto_list.shape[0]
    rows_per_tile = n // NW
    sid = (jnp.arange(n, dtype=jnp.int32) // rows_per_tile) % NS
    labels1d = tf_label_list.astype(jnp.int32) + sid * c
    zeros_acc = jnp.zeros((NS * c, d), jnp.float32)
    zeros_cnt = jnp.zeros((NS * c, 128), jnp.float32)
    ones_cnt = jnp.ones((CHUNK, 128), jnp.float32)
    psums, pcnts = _sc_partial_sums(
        tf_feat_list, labels1d, zeros_acc, zeros_cnt, ones_cnt, c)
    psums = psums.reshape(NW, c, d)
    pcnts = pcnts.reshape(NW, c, 128)
    out = pl.pallas_call(
        _combine_kernel,
        out_shape=jax.ShapeDtypeStruct((1, 1), jnp.float32),
        out_specs=pl.BlockSpec(memory_space=pltpu.SMEM),
    )(psums, pcnts, former_proto_list, former_proto_label.astype(jnp.int32))
    return out[0, 0]


# speedup vs baseline: 8.4347x; 1.0947x over previous
import functools

import jax
import jax.numpy as jnp
from jax import lax
from jax.experimental import pallas as pl
from jax.experimental.pallas import tpu as pltpu
from jax.experimental.pallas import tpu_sc as plsc

NC = 2
NS = 16
NW = NC * NS
CHUNK = 80
HIST_BR = 256


def _sc_partial_sums(feats, labels1d, zeros_acc, c):
    n, d = feats.shape
    nblk = n // CHUNK
    blk_per_tile = nblk // NW
    mesh = plsc.VectorSubcoreMesh(
        core_axis_name="c", subcore_axis_name="s", num_cores=NC, num_subcores=NS)

    @functools.partial(
        pl.kernel,
        out_type=jax.ShapeDtypeStruct((NC, NS * c, d), jnp.float32),
        mesh=mesh,
        scratch_types=[
            pltpu.VMEM((2, CHUNK), jnp.int32),
            pltpu.VMEM((2, CHUNK, d), jnp.float32),
            pltpu.VMEM_SHARED((NS * c, d), jnp.float32),
            pltpu.SemaphoreType.DMA((2,)),
            pltpu.SemaphoreType.DMA((2,)),
        ],
    )
    def run(feats_hbm, labels_hbm, zacc_hbm, out_sums, idx2, buf2,
            shared_acc, lsem, fsem):
        cid = lax.axis_index("c")
        sid = lax.axis_index("s")
        wid = cid * NS + sid
        @pl.when(sid == 0)
        def _():
            pltpu.sync_copy(zacc_hbm, shared_acc)

        plsc.subcore_barrier()

        def start(j):
            p = lax.rem(j, 2)
            blk = wid * blk_per_tile + j
            pltpu.async_copy(
                labels_hbm.at[pl.ds(blk * CHUNK, CHUNK)], idx2.at[p],
                lsem.at[p])
            pltpu.async_copy(
                feats_hbm.at[pl.ds(blk * CHUNK, CHUNK)], buf2.at[p],
                fsem.at[p])

        start(0)

        def body(j, carry):
            p = lax.rem(j, 2)
            pltpu.make_async_copy(
                labels_hbm.at[pl.ds(0, CHUNK)], idx2.at[p], lsem.at[p]).wait()
            pltpu.make_async_copy(
                feats_hbm.at[pl.ds(0, CHUNK)], buf2.at[p], fsem.at[p]).wait()

            @pl.when(j < blk_per_tile - 1)
            def _():
                start(j + 1)

            pltpu.sync_copy(buf2.at[p], shared_acc.at[idx2.at[p]], add=True)
            return carry

        lax.fori_loop(0, blk_per_tile, body, 0)
        plsc.subcore_barrier()

        @pl.when(sid == 0)
        def _():
            pltpu.sync_copy(shared_acc, out_sums.at[cid])

    return run(feats, labels1d, zeros_acc)


def _hist_kernel(lab_ref, out_ref):
    i = pl.program_id(0)
    lab = lab_ref[...]
    class_row = lax.broadcasted_iota(jnp.int32, (HIST_BR, 128), 1)
    acc = jnp.zeros((1, 128), jnp.float32)
    for k in range(128):
        col = jnp.broadcast_to(lab[:, k:k + 1], (HIST_BR, 128))
        oh = (col == class_row).astype(jnp.float32)
        acc = acc + jnp.sum(oh, axis=0, keepdims=True)

    @pl.when(i == 0)
    def _():
        out_ref[...] = jnp.zeros_like(out_ref)

    out_ref[...] += jnp.broadcast_to(acc, out_ref.shape)


def _tc_histogram(labels_pad2d):
    rows = labels_pad2d.shape[0]
    return pl.pallas_call(
        _hist_kernel,
        grid=(rows // HIST_BR,),
        in_specs=[pl.BlockSpec((HIST_BR, 128), lambda i: (i, 0))],
        out_specs=pl.BlockSpec((8, 128), lambda i: (0, 0)),
        out_shape=jax.ShapeDtypeStruct((8, 128), jnp.float32),
    )(labels_pad2d)


def _combine_kernel(psums_ref, hist_ref, proto_ref, plabel_ref, out_ref):
    sums = jnp.sum(psums_ref[...], axis=0)
    c = sums.shape[0]
    d = sums.shape[1]
    labels = plabel_ref[...]
    onehot_c = (labels[:, None] ==
                lax.broadcasted_iota(jnp.int32, (c, c), 1)).astype(jnp.float32)
    onehot_k = (labels[:, None] ==
                lax.broadcasted_iota(jnp.int32, (c, 128), 1)).astype(jnp.float32)
    hist_sq = jnp.broadcast_to(hist_ref[0:1, :], (128, 128))
    eye = (lax.broadcasted_iota(jnp.int32, (128, 128), 0) ==
           lax.broadcasted_iota(jnp.int32, (128, 128), 1))
    hist_col = jnp.sum(jnp.where(eye, hist_sq, 0.0), axis=1,
                       keepdims=True)
    sums_sel = jnp.dot(onehot_c, sums,
                       preferred_element_type=jnp.float32,
                       precision=lax.Precision.HIGHEST)
    cnts_sel = jnp.dot(onehot_k, hist_col,
                       preferred_element_type=jnp.float32,
                       precision=lax.Precision.HIGHEST)
    safe = jnp.maximum(cnts_sel, 1.0)
    proto_cur = sums_sel / safe
    present = (cnts_sel > 0).astype(jnp.float32)
    sq = (proto_ref[...] - proto_cur) ** 2
    per_class = jnp.sum(sq, axis=1, keepdims=True) * present
    denom = jnp.maximum(jnp.sum(present) * d, 1.0)
    out_ref[0, 0] = jnp.sum(per_class) / denom


def kernel(former_proto_list, former_proto_label, tf_feat_list, tf_label_list):
    n, d = tf_feat_list.shape
    c = former_proto_list.shape[0]
    rows_per_tile = n // NW
    labels_i32 = tf_label_list.astype(jnp.int32)
    sid = (jnp.arange(n, dtype=jnp.int32) // rows_per_tile) % NS
    labels1d = labels_i32 + sid * c
    zeros_acc = jnp.zeros((NS * c, d), jnp.float32)
    lrows = n // 128
    lrows_pad = ((lrows + HIST_BR - 1) // HIST_BR) * HIST_BR
    labels_pad = jnp.concatenate(
        [labels_i32, jnp.full((lrows_pad * 128 - n,), 127, jnp.int32)])
    hist = _tc_histogram(labels_pad.reshape(lrows_pad, 128))
    psums = _sc_partial_sums(tf_feat_list, labels1d, zeros_acc, c)
    psums = psums.reshape(NW, c, d)
    out = pl.pallas_call(
        _combine_kernel,
        out_shape=jax.ShapeDtypeStruct((1, 1), jnp.float32),
        out_specs=pl.BlockSpec(memory_space=pltpu.SMEM),
    )(psums, hist, former_proto_list, former_proto_label.astype(jnp.int32))
    return out[0, 0]


# speedup vs baseline: 12.1611x; 1.4418x over previous
import functools

import jax
import jax.numpy as jnp
from jax import lax
from jax.experimental import pallas as pl
from jax.experimental.pallas import tpu as pltpu
from jax.experimental.pallas import tpu_sc as plsc

NC = 2
NS = 16
NW = NC * NS
CHUNK = 80
NBUF = 8
PREFETCH = 4
HIST_BR = 256


def _sc_partial_sums(feats, labels1d, zeros_acc, c):
    n, d = feats.shape
    nblk = n // CHUNK
    blk_per_tile = nblk // NW
    mesh = plsc.VectorSubcoreMesh(
        core_axis_name="c", subcore_axis_name="s", num_cores=NC, num_subcores=NS)

    @functools.partial(
        pl.kernel,
        out_type=jax.ShapeDtypeStruct((NC, NS * c, d), jnp.float32),
        mesh=mesh,
        scratch_types=[
            pltpu.VMEM((NBUF, CHUNK), jnp.int32),
            pltpu.VMEM((NBUF, CHUNK, d), jnp.float32),
            pltpu.VMEM_SHARED((NS * c, d), jnp.float32),
            pltpu.SemaphoreType.DMA((NBUF,)),
            pltpu.SemaphoreType.DMA((NBUF,)),
            pltpu.SemaphoreType.DMA((NBUF,)),
        ],
    )
    def run(feats_hbm, labels_hbm, zacc_hbm, out_sums, idxb, bufb,
            shared_acc, lsem, fsem, ssem):
        cid = lax.axis_index("c")
        sid = lax.axis_index("s")
        wid = cid * NS + sid
        @pl.when(sid == 0)
        def _():
            pltpu.sync_copy(zacc_hbm, shared_acc)

        plsc.subcore_barrier()

        def start(j):
            p = lax.rem(j, NBUF)
            blk = wid * blk_per_tile + j
            pltpu.async_copy(
                labels_hbm.at[pl.ds(blk * CHUNK, CHUNK)], idxb.at[p],
                lsem.at[p])
            pltpu.async_copy(
                feats_hbm.at[pl.ds(blk * CHUNK, CHUNK)], bufb.at[p],
                fsem.at[p])

        def wait_scatter(p):
            pltpu.make_async_copy(
                bufb.at[p], shared_acc.at[idxb.at[p]], ssem.at[p]).wait()

        for j0 in range(PREFETCH):
            start(j0)

        def body(j, carry):
            p = lax.rem(j, NBUF)
            pltpu.make_async_copy(
                labels_hbm.at[pl.ds(0, CHUNK)], idxb.at[p], lsem.at[p]).wait()
            pltpu.make_async_copy(
                feats_hbm.at[pl.ds(0, CHUNK)], bufb.at[p], fsem.at[p]).wait()
            pltpu.async_copy(bufb.at[p], shared_acc.at[idxb.at[p]], ssem.at[p],
                             add=True)
            m = j + PREFETCH

            @pl.when(m < blk_per_tile)
            def _():
                q = lax.rem(m, NBUF)

                @pl.when(m >= NBUF)
                def _():
                    wait_scatter(q)

                start(m)

            return carry

        lax.fori_loop(0, blk_per_tile, body, 0)
        for t in range(NBUF):
            wait_scatter((blk_per_tile - NBUF + t) % NBUF)
        plsc.subcore_barrier()

        @pl.when(sid == 0)
        def _():
            pltpu.sync_copy(shared_acc, out_sums.at[cid])

    return run(feats, labels1d, zeros_acc)


def _hist_kernel(lab_ref, out_ref):
    i = pl.program_id(0)
    lab = lab_ref[...]
    class_row = lax.broadcasted_iota(jnp.int32, (HIST_BR, 128), 1)
    acc = jnp.zeros((1, 128), jnp.float32)
    for k in range(128):
        col = jnp.broadcast_to(lab[:, k:k + 1], (HIST_BR, 128))
        oh = (col == class_row).astype(jnp.float32)
        acc = acc + jnp.sum(oh, axis=0, keepdims=True)

    @pl.when(i == 0)
    def _():
        out_ref[...] = jnp.zeros_like(out_ref)

    out_ref[...] += jnp.broadcast_to(acc, out_ref.shape)


def _tc_histogram(labels_pad2d):
    rows = labels_pad2d.shape[0]
    return pl.pallas_call(
        _hist_kernel,
        grid=(rows // HIST_BR,),
        in_specs=[pl.BlockSpec((HIST_BR, 128), lambda i: (i, 0))],
        out_specs=pl.BlockSpec((8, 128), lambda i: (0, 0)),
        out_shape=jax.ShapeDtypeStruct((8, 128), jnp.float32),
    )(labels_pad2d)


def _combine_kernel(psums_ref, hist_ref, proto_ref, plabel_ref, out_ref):
    sums = jnp.sum(psums_ref[...], axis=0)
    c = sums.shape[0]
    d = sums.shape[1]
    labels = plabel_ref[...]
    onehot_c = (labels[:, None] ==
                lax.broadcasted_iota(jnp.int32, (c, c), 1)).astype(jnp.float32)
    onehot_k = (labels[:, None] ==
                lax.broadcasted_iota(jnp.int32, (c, 128), 1)).astype(jnp.float32)
    hist_sq = jnp.broadcast_to(hist_ref[0:1, :], (128, 128))
    eye = (lax.broadcasted_iota(jnp.int32, (128, 128), 0) ==
           lax.broadcasted_iota(jnp.int32, (128, 128), 1))
    hist_col = jnp.sum(jnp.where(eye, hist_sq, 0.0), axis=1,
                       keepdims=True)
    sums_sel = jnp.dot(onehot_c, sums,
                       preferred_element_type=jnp.float32,
                       precision=lax.Precision.HIGHEST)
    cnts_sel = jnp.dot(onehot_k, hist_col,
                       preferred_element_type=jnp.float32,
                       precision=lax.Precision.HIGHEST)
    safe = jnp.maximum(cnts_sel, 1.0)
    proto_cur = sums_sel / safe
    present = (cnts_sel > 0).astype(jnp.float32)
    sq = (proto_ref[...] - proto_cur) ** 2
    per_class = jnp.sum(sq, axis=1, keepdims=True) * present
    denom = jnp.maximum(jnp.sum(present) * d, 1.0)
    out_ref[0, 0] = jnp.sum(per_class) / denom


def kernel(former_proto_list, former_proto_label, tf_feat_list, tf_label_list):
    n, d = tf_feat_list.shape
    c = former_proto_list.shape[0]
    rows_per_tile = n // NW
    labels_i32 = tf_label_list.astype(jnp.int32)
    sid = (jnp.arange(n, dtype=jnp.int32) // rows_per_tile) % NS
    labels1d = labels_i32 + sid * c
    zeros_acc = jnp.zeros((NS * c, d), jnp.float32)
    lrows = n // 128
    lrows_pad = ((lrows + HIST_BR - 1) // HIST_BR) * HIST_BR
    labels_pad = jnp.concatenate(
        [labels_i32, jnp.full((lrows_pad * 128 - n,), 127, jnp.int32)])
    hist = _tc_histogram(labels_pad.reshape(lrows_pad, 128))
    psums = _sc_partial_sums(tf_feat_list, labels1d, zeros_acc, c)
    psums = psums.reshape(NW, c, d)
    out = pl.pallas_call(
        _combine_kernel,
        out_shape=jax.ShapeDtypeStruct((1, 1), jnp.float32),
        out_specs=pl.BlockSpec(memory_space=pltpu.SMEM),
    )(psums, hist, former_proto_list, former_proto_label.astype(jnp.int32))
    return out[0, 0]


# speedup vs baseline: 12.7142x; 1.0455x over previous
import functools

import jax
import jax.numpy as jnp
from jax import lax
from jax.experimental import pallas as pl
from jax.experimental.pallas import tpu as pltpu
from jax.experimental.pallas import tpu_sc as plsc

NC = 2
NS = 16
NW = NC * NS
CHUNK = 80
NBUF = 10
PREFETCH = 5
HIST_BR = 256


def _sc_partial_sums(feats, labels1d, zeros_acc, c):
    n, d = feats.shape
    nblk = n // CHUNK
    blk_per_tile = nblk // NW
    mesh = plsc.VectorSubcoreMesh(
        core_axis_name="c", subcore_axis_name="s", num_cores=NC, num_subcores=NS)

    @functools.partial(
        pl.kernel,
        out_type=jax.ShapeDtypeStruct((NC, NS * c, d), jnp.float32),
        mesh=mesh,
        scratch_types=[
            pltpu.VMEM((NBUF, CHUNK), jnp.int32),
            pltpu.VMEM((NBUF, CHUNK, d), jnp.float32),
            pltpu.VMEM_SHARED((NS * c, d), jnp.float32),
            pltpu.SemaphoreType.DMA((NBUF,)),
            pltpu.SemaphoreType.DMA((NBUF,)),
            pltpu.SemaphoreType.DMA((NBUF,)),
        ],
    )
    def run(feats_hbm, labels_hbm, zacc_hbm, out_sums, idxb, bufb,
            shared_acc, lsem, fsem, ssem):
        cid = lax.axis_index("c")
        sid = lax.axis_index("s")
        wid = cid * NS + sid
        @pl.when(sid == 0)
        def _():
            pltpu.sync_copy(zacc_hbm, shared_acc)

        plsc.subcore_barrier()

        def start(j):
            p = lax.rem(j, NBUF)
            blk = wid * blk_per_tile + j
            pltpu.async_copy(
                labels_hbm.at[pl.ds(blk * CHUNK, CHUNK)], idxb.at[p],
                lsem.at[p])
            pltpu.async_copy(
                feats_hbm.at[pl.ds(blk * CHUNK, CHUNK)], bufb.at[p],
                fsem.at[p])

        def wait_scatter(p):
            pltpu.make_async_copy(
                bufb.at[p], shared_acc.at[idxb.at[p]], ssem.at[p]).wait()

        for j0 in range(PREFETCH):
            start(j0)

        def body(j, carry):
            p = lax.rem(j, NBUF)
            pltpu.make_async_copy(
                labels_hbm.at[pl.ds(0, CHUNK)], idxb.at[p], lsem.at[p]).wait()
            pltpu.make_async_copy(
                feats_hbm.at[pl.ds(0, CHUNK)], bufb.at[p], fsem.at[p]).wait()
            pltpu.async_copy(bufb.at[p], shared_acc.at[idxb.at[p]], ssem.at[p],
                             add=True)
            m = j + PREFETCH

            @pl.when(m < blk_per_tile)
            def _():
                q = lax.rem(m, NBUF)

                @pl.when(m >= NBUF)
                def _():
                    wait_scatter(q)

                start(m)

            return carry

        lax.fori_loop(0, blk_per_tile, body, 0)
        for t in range(NBUF):
            wait_scatter((blk_per_tile - NBUF + t) % NBUF)
        plsc.subcore_barrier()

        @pl.when(sid == 0)
        def _():
            pltpu.sync_copy(shared_acc, out_sums.at[cid])

    return run(feats, labels1d, zeros_acc)


def _hist_kernel(lab_ref, out_ref):
    i = pl.program_id(0)
    lab = lab_ref[...]
    class_row = lax.broadcasted_iota(jnp.int32, (HIST_BR, 128), 1)
    acc = jnp.zeros((1, 128), jnp.float32)
    for k in range(128):
        col = jnp.broadcast_to(lab[:, k:k + 1], (HIST_BR, 128))
        oh = (col == class_row).astype(jnp.float32)
        acc = acc + jnp.sum(oh, axis=0, keepdims=True)

    @pl.when(i == 0)
    def _():
        out_ref[...] = jnp.zeros_like(out_ref)

    out_ref[...] += jnp.broadcast_to(acc, out_ref.shape)


def _tc_histogram(labels_pad2d):
    rows = labels_pad2d.shape[0]
    return pl.pallas_call(
        _hist_kernel,
        grid=(rows // HIST_BR,),
        in_specs=[pl.BlockSpec((HIST_BR, 128), lambda i: (i, 0))],
        out_specs=pl.BlockSpec((8, 128), lambda i: (0, 0)),
        out_shape=jax.ShapeDtypeStruct((8, 128), jnp.float32),
    )(labels_pad2d)


def _combine_kernel(psums_ref, hist_ref, proto_ref, plabel_ref, out_ref):
    sums = jnp.sum(psums_ref[...], axis=0)
    c = sums.shape[0]
    d = sums.shape[1]
    labels = plabel_ref[...]
    onehot_c = (labels[:, None] ==
                lax.broadcasted_iota(jnp.int32, (c, c), 1)).astype(jnp.float32)
    onehot_k = (labels[:, None] ==
                lax.broadcasted_iota(jnp.int32, (c, 128), 1)).astype(jnp.float32)
    hist_sq = jnp.broadcast_to(hist_ref[0:1, :], (128, 128))
    eye = (lax.broadcasted_iota(jnp.int32, (128, 128), 0) ==
           lax.broadcasted_iota(jnp.int32, (128, 128), 1))
    hist_col = jnp.sum(jnp.where(eye, hist_sq, 0.0), axis=1,
                       keepdims=True)
    sums_sel = jnp.dot(onehot_c, sums,
                       preferred_element_type=jnp.float32,
                       precision=lax.Precision.HIGHEST)
    cnts_sel = jnp.dot(onehot_k, hist_col,
                       preferred_element_type=jnp.float32,
                       precision=lax.Precision.HIGHEST)
    safe = jnp.maximum(cnts_sel, 1.0)
    proto_cur = sums_sel / safe
    present = (cnts_sel > 0).astype(jnp.float32)
    sq = (proto_ref[...] - proto_cur) ** 2
    per_class = jnp.sum(sq, axis=1, keepdims=True) * present
    denom = jnp.maximum(jnp.sum(present) * d, 1.0)
    out_ref[0, 0] = jnp.sum(per_class) / denom


def kernel(former_proto_list, former_proto_label, tf_feat_list, tf_label_list):
    n, d = tf_feat_list.shape
    c = former_proto_list.shape[0]
    rows_per_tile = n // NW
    labels_i32 = tf_label_list.astype(jnp.int32)
    sid = (jnp.arange(n, dtype=jnp.int32) // rows_per_tile) % NS
    labels1d = labels_i32 + sid * c
    zeros_acc = jnp.zeros((NS * c, d), jnp.float32)
    lrows = n // 128
    lrows_pad = ((lrows + HIST_BR - 1) // HIST_BR) * HIST_BR
    labels_pad = jnp.concatenate(
        [labels_i32, jnp.full((lrows_pad * 128 - n,), 127, jnp.int32)])
    hist = _tc_histogram(labels_pad.reshape(lrows_pad, 128))
    psums = _sc_partial_sums(tf_feat_list, labels1d, zeros_acc, c)
    psums = psums.reshape(NW, c, d)
    out = pl.pallas_call(
        _combine_kernel,
        out_shape=jax.ShapeDtypeStruct((1, 1), jnp.float32),
        out_specs=pl.BlockSpec(memory_space=pltpu.SMEM),
    )(psums, hist, former_proto_list, former_proto_label.astype(jnp.int32))
    return out[0, 0]
